# baseline (speedup 1.0000x reference)
import functools

import jax
import jax.numpy as jnp
from jax import lax
from jax.experimental import pallas as pl
from jax.experimental.pallas import tpu as pltpu
from jax.experimental.pallas import tpu_sc as plsc

_L = 16
_NW = 32
_CH = 128

_LOG_COEFFS = (
    7.0376836292e-2, -1.1514610310e-1, 1.1676998740e-1,
    -1.2420140846e-1, 1.4249322787e-1, -1.6668057665e-1,
    1.9999714748e-1, -2.4999993993e-1, 3.3333331174e-1,
)
_LN2 = 0.6931471805599453
_SQRT2 = 1.41421356


def _softplus16(x):
    e = jnp.exp(jnp.minimum(x, 20.0))
    t = 1.0 + e
    i = lax.bitcast_convert_type(t, jnp.int32)
    ex = lax.shift_right_logical(i, 23) - 127
    m = lax.bitcast_convert_type((i & 0x7FFFFF) | 0x3F800000, jnp.float32)
    big = m > _SQRT2
    m = jnp.where(big, m * 0.5, m)
    exf = ex.astype(jnp.float32) + jnp.where(big, 1.0, 0.0)
    f = m - 1.0
    z = f * f
    p = jnp.full_like(f, _LOG_COEFFS[0])
    for c in _LOG_COEFFS[1:]:
        p = p * f + c
    logt = f * z * p - 0.5 * z + f + exf * _LN2
    return jnp.where(x > 20.0, x, logt)


def kernel(idxs, U, V):
    B = idxs.shape[0]
    D = U.shape[1]
    R = 2 * B
    rows_per_w = R // _NW
    nch = rows_per_w // _CH
    idx3 = idxs.astype(jnp.int32).reshape(_NW, nch, _CH)

    mesh = plsc.VectorSubcoreMesh(core_axis_name="c", subcore_axis_name="s")

    @functools.partial(
        pl.kernel,
        out_type=jax.ShapeDtypeStruct((2 * R, D), jnp.float32),
        mesh=mesh,
        compiler_params=pltpu.CompilerParams(use_tc_tiling_on_sc=False),
        scratch_types=[
            pltpu.VMEM((nch, _CH), jnp.int32),
            pltpu.VMEM((_CH, D), jnp.float32),
            pltpu.VMEM((_CH, D), jnp.float32),
            pltpu.VMEM((2 * _CH, D), jnp.float32),
            pltpu.SemaphoreType.DMA,
            pltpu.SemaphoreType.DMA,
            pltpu.SemaphoreType.DMA,
        ],
    )
    def run(idx_hbm, u_hbm, v_hbm, out_hbm, idx_v, ub, vb, ob, su, sv, so):
        wid = lax.axis_index("s") * 2 + lax.axis_index("c")
        pltpu.sync_copy(idx_hbm.at[wid], idx_v)
        for c in range(nch):
            cu = pltpu.async_copy(u_hbm.at[idx_v.at[c]], ub, su)
            cv = pltpu.async_copy(v_hbm.at[idx_v.at[c]], vb, sv)
            cu.wait()
            cv.wait()

            def body(r, _):
                for l in range(D // _L):
                    s = pl.ds(l * _L, _L)
                    ob[2 * r, s] = ub[r, s]
                    ob[2 * r + 1, s] = _softplus16(vb[r, s])
                return 0

            lax.fori_loop(0, _CH, body, 0)
            out_base = 2 * (wid * rows_per_w + c * _CH)
            pltpu.async_copy(ob, out_hbm.at[pl.ds(out_base, 2 * _CH)], so).wait()

    out = run(idx3, U, V)
    return out.reshape(B, 2, 2, D)


# speedup vs baseline: 1.8321x; 1.8321x over previous
import functools

import jax
import jax.numpy as jnp
from jax import lax
from jax.experimental import pallas as pl
from jax.experimental.pallas import tpu as pltpu
from jax.experimental.pallas import tpu_sc as plsc

_L = 16
_NW = 32
_CH = 16

_LOG_COEFFS = (
    7.0376836292e-2, -1.1514610310e-1, 1.1676998740e-1,
    -1.2420140846e-1, 1.4249322787e-1, -1.6668057665e-1,
    1.9999714748e-1, -2.4999993993e-1, 3.3333331174e-1,
)
_LN2 = 0.6931471805599453
_SQRT2 = 1.41421356


def _softplus16(x):
    e = jnp.exp(jnp.minimum(x, 20.0))
    t = 1.0 + e
    i = lax.bitcast_convert_type(t, jnp.int32)
    ex = lax.shift_right_logical(i, 23) - 127
    m = lax.bitcast_convert_type((i & 0x7FFFFF) | 0x3F800000, jnp.float32)
    big = m > _SQRT2
    m = jnp.where(big, m * 0.5, m)
    exf = ex.astype(jnp.float32) + jnp.where(big, 1.0, 0.0)
    f = m - 1.0
    z = f * f
    p = jnp.full_like(f, _LOG_COEFFS[0])
    for c in _LOG_COEFFS[1:]:
        p = p * f + c
    logt = f * z * p - 0.5 * z + f + exf * _LN2
    return jnp.where(x > 20.0, x, logt)


def kernel(idxs, U, V):
    B = idxs.shape[0]
    D = U.shape[1]
    R = 2 * B
    rows_per_w = R // _NW
    nch = rows_per_w // _CH
    idx_i32 = idxs.astype(jnp.int32).reshape(_NW, rows_per_w)
    U3 = U.reshape(U.shape[0] // 8, 8, D)
    V3 = V.reshape(V.shape[0] // 8, 8, D)

    mesh = plsc.VectorSubcoreMesh(core_axis_name="c", subcore_axis_name="s")

    @functools.partial(
        pl.kernel,
        out_type=jax.ShapeDtypeStruct((R, 2 * D), jnp.float32),
        mesh=mesh,
        compiler_params=pltpu.CompilerParams(use_tc_tiling_on_sc=True),
        scratch_types=[
            pltpu.VMEM((rows_per_w + _L,), jnp.int32),
            pltpu.VMEM((_CH, 8, D), jnp.float32),
            pltpu.VMEM((_CH, 8, D), jnp.float32),
            pltpu.VMEM((_CH, 8, D), jnp.float32),
            pltpu.VMEM((_CH, 8, D), jnp.float32),
            pltpu.VMEM((_CH, 2 * D), jnp.float32),
            pltpu.VMEM((_CH, 2 * D), jnp.float32),
            pltpu.SemaphoreType.DMA,
            pltpu.SemaphoreType.DMA,
            pltpu.SemaphoreType.DMA,
            pltpu.SemaphoreType.DMA,
        ],
    )
    def run(idx_hbm, u_hbm, v_hbm, out_hbm,
            idx_v, ubA, vbA, ubB, vbB, obA, obB, semA, semB, soA, soB):
        wid = lax.axis_index("s") * 2 + lax.axis_index("c")
        pltpu.sync_copy(idx_hbm.at[wid], idx_v.at[pl.ds(0, rows_per_w)])

        def fire(c, ub, vb, sem):
            def fire_body(r, _):
                k = idx_v[pl.ds(c * _CH + r, _L)][0]
                pltpu.async_copy(u_hbm.at[k >> 3], ub.at[r], sem)
                pltpu.async_copy(v_hbm.at[k >> 3], vb.at[r], sem)
                return 0

            lax.fori_loop(0, _CH, fire_body, 0)

        def consume(c, ub, vb, sem, ob, so):
            pltpu.make_async_copy(u_hbm.at[pl.ds(0, _CH)], ub, sem).wait()
            pltpu.make_async_copy(v_hbm.at[pl.ds(0, _CH)], vb, sem).wait()

            def row_body(r, _):
                rit = idx_v[pl.ds(c * _CH + r, _L)][0] & 7
                for l in range(D // _L):
                    ob[r, pl.ds(l * _L, _L)] = ub[r, rit, pl.ds(l * _L, _L)]
                    ob[r, pl.ds(D + l * _L, _L)] = _softplus16(
                        vb[r, rit, pl.ds(l * _L, _L)])
                return 0

            lax.fori_loop(0, _CH, row_body, 0)
            out_base = wid * rows_per_w + c * _CH
            pltpu.async_copy(ob, out_hbm.at[pl.ds(out_base, _CH)], so).wait()

        fire(0, ubA, vbA, semA)

        def pair_body(g, _):
            c0 = 2 * g
            fire(c0 + 1, ubB, vbB, semB)
            consume(c0, ubA, vbA, semA, obA, soA)

            @pl.when(c0 + 2 < nch)
            def _():
                fire(c0 + 2, ubA, vbA, semA)

            consume(c0 + 1, ubB, vbB, semB, obB, soB)
            return 0

        lax.fori_loop(0, nch // 2, pair_body, 0)

    out = run(idx_i32, U3, V3)
    return out.reshape(B, 2, 2, D)
